# baseline (speedup 1.0000x reference)
import functools

import jax
import jax.numpy as jnp
from jax import lax
from jax.experimental import pallas as pl
from jax.experimental.pallas import tpu as pltpu


TM = 128


def _gemm_tile(te_ref, x_ref, w1_ref, w2_ref, wt_ref, y_ref, *, n_inter):
    x = x_ref[...].astype(jnp.bfloat16)
    w1 = w1_ref[0].astype(jnp.bfloat16)
    h = jnp.dot(x, w1, preferred_element_type=jnp.float32)
    gate = h[:, :n_inter]
    up = h[:, n_inter:]
    act = (gate * jax.nn.sigmoid(gate)) * up
    w2 = w2_ref[0].astype(jnp.bfloat16)
    y = jnp.dot(act.astype(jnp.bfloat16), w2,
                preferred_element_type=jnp.float32)
    y_ref[...] = y * wt_ref[...]


def _grouped_gemm(x_rows, w1, w2, row_weight, tile_expert):
    p_pad, k_dim = x_rows.shape
    e_dim, _, n2 = w1.shape
    n_inter = w2.shape[1]
    tiles = p_pad // TM
    grid_spec = pltpu.PrefetchScalarGridSpec(
        num_scalar_prefetch=1,
        grid=(tiles,),
        in_specs=[
            pl.BlockSpec((TM, k_dim), lambda t, te: (t, 0)),
            pl.BlockSpec((1, k_dim, n2), lambda t, te: (te[t], 0, 0)),
            pl.BlockSpec((1, n_inter, k_dim), lambda t, te: (te[t], 0, 0)),
            pl.BlockSpec((TM, 1), lambda t, te: (t, 0)),
        ],
        out_specs=pl.BlockSpec((TM, k_dim), lambda t, te: (t, 0)),
    )
    return pl.pallas_call(
        functools.partial(_gemm_tile, n_inter=n_inter),
        grid_spec=grid_spec,
        out_shape=jax.ShapeDtypeStruct((p_pad, k_dim), jnp.float32),
    )(tile_expert, x_rows, w1, w2, row_weight)


def kernel(hidden_states, w1, w2, topk_weights, topk_ids):
    m, k_dim = hidden_states.shape
    e_dim = w1.shape[0]
    topk = topk_ids.shape[1]
    p = m * topk
    p_pad = p + e_dim * TM

    e_flat = topk_ids.reshape(p)
    order = jnp.argsort(e_flat, stable=True)
    e_sorted = e_flat[order]
    counts = jnp.zeros(e_dim, jnp.int32).at[e_flat].add(1)
    start = jnp.cumsum(counts) - counts
    pg = ((counts + TM - 1) // TM) * TM
    pg_cum = jnp.cumsum(pg)
    pstart = pg_cum - pg
    rank = jnp.arange(p, dtype=jnp.int32) - start[e_sorted]
    dest = (pstart[e_sorted] + rank).astype(jnp.int32)
    row_token = jnp.zeros(p_pad, jnp.int32).at[dest].set(
        (order // topk).astype(jnp.int32))
    row_weight = jnp.zeros((p_pad, 1), jnp.float32).at[dest, 0].set(
        topk_weights.reshape(p)[order])
    row_of_pair = jnp.zeros(p, jnp.int32).at[order].set(dest)
    tiles = p_pad // TM
    tile_expert = jnp.minimum(
        jnp.searchsorted(pg_cum, jnp.arange(tiles, dtype=jnp.int32) * TM,
                         side='right'),
        e_dim - 1).astype(jnp.int32)

    x_rows = hidden_states[row_token]

    y_rows = _grouped_gemm(x_rows, w1, w2, row_weight, tile_expert)

    out = y_rows[row_of_pair].reshape(m, topk, k_dim).sum(axis=1)
    return out


# speedup vs baseline: 1.2247x; 1.2247x over previous
import functools

import jax
import jax.numpy as jnp
from jax import lax
from jax.experimental import pallas as pl
from jax.experimental.pallas import tpu as pltpu


TM = 128


def _gemm_tile(te_ref, x_ref, w1_ref, w2_ref, wt_ref, y_ref, *, n_inter):
    x = x_ref[...].astype(jnp.bfloat16)
    w1 = w1_ref[0].astype(jnp.bfloat16)
    h = jnp.dot(x, w1, preferred_element_type=jnp.float32)
    gate = h[:, :n_inter]
    up = h[:, n_inter:]
    act = (gate * jax.nn.sigmoid(gate)) * up
    w2 = w2_ref[0].astype(jnp.bfloat16)
    y = jnp.dot(act.astype(jnp.bfloat16), w2,
                preferred_element_type=jnp.float32)
    y_ref[...] = y * wt_ref[...]


def _grouped_gemm(x_rows, w1, w2, row_weight, tile_expert):
    p_pad, k_dim = x_rows.shape
    e_dim, _, n2 = w1.shape
    n_inter = w2.shape[1]
    tiles = p_pad // TM
    grid_spec = pltpu.PrefetchScalarGridSpec(
        num_scalar_prefetch=1,
        grid=(tiles,),
        in_specs=[
            pl.BlockSpec((TM, k_dim), lambda t, te: (t, 0)),
            pl.BlockSpec((1, k_dim, n2), lambda t, te: (te[t], 0, 0)),
            pl.BlockSpec((1, n_inter, k_dim), lambda t, te: (te[t], 0, 0)),
            pl.BlockSpec((TM, 1), lambda t, te: (t, 0)),
        ],
        out_specs=pl.BlockSpec((TM, k_dim), lambda t, te: (t, 0)),
    )
    return pl.pallas_call(
        functools.partial(_gemm_tile, n_inter=n_inter),
        grid_spec=grid_spec,
        out_shape=jax.ShapeDtypeStruct((p_pad, k_dim), jnp.float32),
    )(tile_expert, x_rows, w1, w2, row_weight)


def kernel(hidden_states, w1, w2, topk_weights, topk_ids):
    m, k_dim = hidden_states.shape
    e_dim = w1.shape[0]
    topk = topk_ids.shape[1]
    p = m * topk
    p_pad = p + e_dim * TM

    e_flat = topk_ids.reshape(p)
    onehot = (e_flat[None, :] == jnp.arange(e_dim, dtype=jnp.int32)[:, None])
    onehot = onehot.astype(jnp.int32)
    csum = jnp.cumsum(onehot, axis=1)
    counts = csum[:, -1]
    rank = jnp.sum(onehot * csum, axis=0) - 1
    pg = ((counts + TM - 1) // TM) * TM
    pg_cum = jnp.cumsum(pg)
    pstart = pg_cum - pg
    dest = (pstart[e_flat] + rank).astype(jnp.int32)
    row_token = jnp.zeros(p_pad, jnp.int32).at[dest].set(
        (jnp.arange(p, dtype=jnp.int32) // topk))
    row_weight = jnp.zeros((p_pad, 1), jnp.float32).at[dest, 0].set(
        topk_weights.reshape(p))
    row_of_pair = dest
    tiles = p_pad // TM
    tile_expert = jnp.sum(
        (jnp.arange(tiles, dtype=jnp.int32)[:, None] * TM >= pg_cum[None, :])
        .astype(jnp.int32), axis=1)
    tile_expert = jnp.minimum(tile_expert, e_dim - 1).astype(jnp.int32)

    x_rows = hidden_states[row_token]

    y_rows = _grouped_gemm(x_rows, w1, w2, row_weight, tile_expert)

    out = y_rows[row_of_pair].reshape(m, topk, k_dim).sum(axis=1)
    return out


# speedup vs baseline: 1.4803x; 1.2087x over previous
import functools

import jax
import jax.numpy as jnp
from jax import lax
from jax.experimental import pallas as pl
from jax.experimental.pallas import tpu as pltpu
from jax.experimental.pallas import tpu_sc as plsc


TM = 128
NC = 2
NS = 16
NW = NC * NS
LANES = 16


def _wid():
    return lax.axis_index("s") * NC + lax.axis_index("c")


def _dispatch_body(idx_hbm, hid_hbm, out_hbm, idx_v, buf, sem, *, rows_w):
    base = _wid() * rows_w
    pltpu.sync_copy(idx_hbm.at[pl.ds(base, rows_w)], idx_v)
    nch = (rows_w + 127) // 128
    ch = rows_w // nch
    cps = [
        pltpu.async_copy(hid_hbm.at[idx_v.at[pl.ds(c * ch, ch)]],
                         buf.at[pl.ds(c * ch, ch)], sem)
        for c in range(nch)
    ]
    for cp in cps:
        cp.wait()
    pltpu.sync_copy(buf, out_hbm.at[pl.ds(base, rows_w)])


def _sc_dispatch(row_token, hidden):
    p_pad = row_token.shape[0]
    k_dim = hidden.shape[1]
    rows_w = p_pad // NW
    mesh = plsc.VectorSubcoreMesh(core_axis_name="c", subcore_axis_name="s")
    f = pl.kernel(
        functools.partial(_dispatch_body, rows_w=rows_w),
        out_type=jax.ShapeDtypeStruct((p_pad, k_dim), jnp.float32),
        mesh=mesh,
        scratch_types=[
            pltpu.VMEM((rows_w,), jnp.int32),
            pltpu.VMEM((rows_w, k_dim), jnp.float32),
            pltpu.SemaphoreType.DMA,
        ],
    )
    return f(row_token, hidden)


def _finalize_body(ridx_hbm, y_hbm, out_hbm, idx0, idx1, b0, b1, s0, s1,
                   *, tok_w, k_dim, topk):
    base = _wid() * tok_w
    pltpu.sync_copy(ridx_hbm.at[0, pl.ds(base, tok_w)], idx0)
    cp0 = pltpu.async_copy(y_hbm.at[idx0], b0, s0)
    for s in range(1, topk):
        pltpu.sync_copy(ridx_hbm.at[s, pl.ds(base, tok_w)], idx1)
        cp1 = pltpu.async_copy(y_hbm.at[idx1], b1, s1)
        if s == 1:
            cp0.wait()
        cp1.wait()

        def row_body(i, carry):
            for j in range(k_dim // LANES):
                sl = pl.ds(j * LANES, LANES)
                plsc.addupdate(b0.at[i, sl], b1[i, sl])
            return carry

        lax.fori_loop(0, tok_w, row_body, 0)
    pltpu.sync_copy(b0, out_hbm.at[pl.ds(base, tok_w)])


def _sc_finalize(ridx, y_rows, m):
    topk = ridx.shape[0]
    k_dim = y_rows.shape[1]
    tok_w = m // NW
    mesh = plsc.VectorSubcoreMesh(core_axis_name="c", subcore_axis_name="s")
    f = pl.kernel(
        functools.partial(_finalize_body, tok_w=tok_w, k_dim=k_dim, topk=topk),
        out_type=jax.ShapeDtypeStruct((m, k_dim), jnp.float32),
        mesh=mesh,
        scratch_types=[
            pltpu.VMEM((tok_w,), jnp.int32),
            pltpu.VMEM((tok_w,), jnp.int32),
            pltpu.VMEM((tok_w, k_dim), jnp.float32),
            pltpu.VMEM((tok_w, k_dim), jnp.float32),
            pltpu.SemaphoreType.DMA,
            pltpu.SemaphoreType.DMA,
        ],
    )
    return f(ridx, y_rows)


def _gemm_tile(te_ref, x_ref, w1_ref, w2_ref, wt_ref, y_ref, *, n_inter):
    x = x_ref[...].astype(jnp.bfloat16)
    w1 = w1_ref[0].astype(jnp.bfloat16)
    h = jnp.dot(x, w1, preferred_element_type=jnp.float32)
    gate = h[:, :n_inter]
    up = h[:, n_inter:]
    act = (gate * jax.nn.sigmoid(gate)) * up
    w2 = w2_ref[0].astype(jnp.bfloat16)
    y = jnp.dot(act.astype(jnp.bfloat16), w2,
                preferred_element_type=jnp.float32)
    y_ref[...] = y * wt_ref[...]


def _grouped_gemm(x_rows, w1, w2, row_weight, tile_expert):
    p_pad, k_dim = x_rows.shape
    e_dim, _, n2 = w1.shape
    n_inter = w2.shape[1]
    tiles = p_pad // TM
    grid_spec = pltpu.PrefetchScalarGridSpec(
        num_scalar_prefetch=1,
        grid=(tiles,),
        in_specs=[
            pl.BlockSpec((TM, k_dim), lambda t, te: (t, 0)),
            pl.BlockSpec((1, k_dim, n2), lambda t, te: (te[t], 0, 0)),
            pl.BlockSpec((1, n_inter, k_dim), lambda t, te: (te[t], 0, 0)),
            pl.BlockSpec((TM, 1), lambda t, te: (t, 0)),
        ],
        out_specs=pl.BlockSpec((TM, k_dim), lambda t, te: (t, 0)),
    )
    return pl.pallas_call(
        functools.partial(_gemm_tile, n_inter=n_inter),
        grid_spec=grid_spec,
        out_shape=jax.ShapeDtypeStruct((p_pad, k_dim), jnp.float32),
    )(tile_expert, x_rows, w1, w2, row_weight)


def kernel(hidden_states, w1, w2, topk_weights, topk_ids):
    m, k_dim = hidden_states.shape
    e_dim = w1.shape[0]
    topk = topk_ids.shape[1]
    p = m * topk
    p_pad = p + e_dim * TM

    e_flat = topk_ids.reshape(p)
    onehot = (e_flat[None, :] == jnp.arange(e_dim, dtype=jnp.int32)[:, None])
    onehot = onehot.astype(jnp.int32)
    csum = jnp.cumsum(onehot, axis=1)
    counts = csum[:, -1]
    rank = jnp.sum(onehot * csum, axis=0) - 1
    pg = ((counts + TM - 1) // TM) * TM
    pg_cum = jnp.cumsum(pg)
    pstart = pg_cum - pg
    dest = (pstart[e_flat] + rank).astype(jnp.int32)
    row_token = jnp.zeros(p_pad, jnp.int32).at[dest].set(
        (jnp.arange(p, dtype=jnp.int32) // topk))
    row_weight = jnp.zeros((p_pad, 1), jnp.float32).at[dest, 0].set(
        topk_weights.reshape(p))
    row_of_pair = dest
    tiles = p_pad // TM
    tile_expert = jnp.sum(
        (jnp.arange(tiles, dtype=jnp.int32)[:, None] * TM >= pg_cum[None, :])
        .astype(jnp.int32), axis=1)
    tile_expert = jnp.minimum(tile_expert, e_dim - 1).astype(jnp.int32)

    x_rows = _sc_dispatch(row_token, hidden_states)

    y_rows = _grouped_gemm(x_rows, w1, w2, row_weight, tile_expert)

    ridx = row_of_pair.reshape(m, topk).T
    out = _sc_finalize(ridx, y_rows, m)
    return out


# speedup vs baseline: 1.9059x; 1.2875x over previous
import functools

import jax
import jax.numpy as jnp
from jax import lax
from jax.experimental import pallas as pl
from jax.experimental.pallas import tpu as pltpu
from jax.experimental.pallas import tpu_sc as plsc


TM = 128
NC = 2
NS = 16
NW = NC * NS
LANES = 16


def _wid():
    return lax.axis_index("s") * NC + lax.axis_index("c")


def _dispatch_body(idx_hbm, hid_hbm, out_hbm, idx_v, buf, sem, *, rows_w):
    base = _wid() * rows_w
    pltpu.sync_copy(idx_hbm.at[pl.ds(base, rows_w)], idx_v)
    nch = (rows_w + 127) // 128
    ch = rows_w // nch
    cps = [
        pltpu.async_copy(hid_hbm.at[idx_v.at[pl.ds(c * ch, ch)]],
                         buf.at[pl.ds(c * ch, ch)], sem)
        for c in range(nch)
    ]
    for cp in cps:
        cp.wait()
    pltpu.sync_copy(buf, out_hbm.at[pl.ds(base, rows_w)])


def _sc_dispatch(row_token, hidden):
    p_pad = row_token.shape[0]
    k_dim = hidden.shape[1]
    rows_w = p_pad // NW
    mesh = plsc.VectorSubcoreMesh(core_axis_name="c", subcore_axis_name="s")
    f = pl.kernel(
        functools.partial(_dispatch_body, rows_w=rows_w),
        out_type=jax.ShapeDtypeStruct((p_pad, k_dim), jnp.float32),
        mesh=mesh,
        scratch_types=[
            pltpu.VMEM((rows_w,), jnp.int32),
            pltpu.VMEM((rows_w, k_dim), jnp.float32),
            pltpu.SemaphoreType.DMA,
        ],
    )
    return f(row_token, hidden)


def _finalize_body(ridx_hbm, y_hbm, out_hbm, idx0, idx1, b0, b1, s0, s1,
                   *, tok_w, k_dim, topk):
    base = _wid() * tok_w
    pltpu.sync_copy(ridx_hbm.at[0, pl.ds(base, tok_w)], idx0)
    cp0 = pltpu.async_copy(y_hbm.at[idx0], b0, s0)
    for s in range(1, topk):
        pltpu.sync_copy(ridx_hbm.at[s, pl.ds(base, tok_w)], idx1)
        cp1 = pltpu.async_copy(y_hbm.at[idx1], b1, s1)
        if s == 1:
            cp0.wait()
        cp1.wait()

        def row_body(i, carry):
            for j in range(k_dim // LANES):
                sl = pl.ds(j * LANES, LANES)
                plsc.addupdate(b0.at[i, sl], b1[i, sl])
            return carry

        lax.fori_loop(0, tok_w, row_body, 0)
    pltpu.sync_copy(b0, out_hbm.at[pl.ds(base, tok_w)])


def _sc_finalize(ridx, y_rows, m):
    topk = ridx.shape[0]
    k_dim = y_rows.shape[1]
    tok_w = m // NW
    mesh = plsc.VectorSubcoreMesh(core_axis_name="c", subcore_axis_name="s")
    f = pl.kernel(
        functools.partial(_finalize_body, tok_w=tok_w, k_dim=k_dim, topk=topk),
        out_type=jax.ShapeDtypeStruct((m, k_dim), jnp.float32),
        mesh=mesh,
        scratch_types=[
            pltpu.VMEM((tok_w,), jnp.int32),
            pltpu.VMEM((tok_w,), jnp.int32),
            pltpu.VMEM((tok_w, k_dim), jnp.float32),
            pltpu.VMEM((tok_w, k_dim), jnp.float32),
            pltpu.SemaphoreType.DMA,
            pltpu.SemaphoreType.DMA,
        ],
    )
    return f(ridx, y_rows)


def _gemm_tile(te_ref, x_ref, w1_ref, w2_ref, wt_ref, y_ref, *, n_inter):
    x = x_ref[...].astype(jnp.bfloat16)
    w1 = w1_ref[0].astype(jnp.bfloat16)
    h = jnp.dot(x, w1, preferred_element_type=jnp.float32)
    gate = h[:, :n_inter]
    up = h[:, n_inter:]
    act = (gate * jax.nn.sigmoid(gate)) * up
    w2 = w2_ref[0].astype(jnp.bfloat16)
    y = jnp.dot(act.astype(jnp.bfloat16), w2,
                preferred_element_type=jnp.float32)
    y_ref[...] = y * wt_ref[...]


def _grouped_gemm(x_rows, w1, w2, row_weight, tile_expert):
    p_pad, k_dim = x_rows.shape
    e_dim, _, n2 = w1.shape
    n_inter = w2.shape[1]
    tiles = p_pad // TM
    grid_spec = pltpu.PrefetchScalarGridSpec(
        num_scalar_prefetch=1,
        grid=(tiles,),
        in_specs=[
            pl.BlockSpec((TM, k_dim), lambda t, te: (t, 0)),
            pl.BlockSpec((1, k_dim, n2), lambda t, te: (te[t], 0, 0)),
            pl.BlockSpec((1, n_inter, k_dim), lambda t, te: (te[t], 0, 0)),
            pl.BlockSpec((TM, 1), lambda t, te: (t, 0)),
        ],
        out_specs=pl.BlockSpec((TM, k_dim), lambda t, te: (t, 0)),
    )
    return pl.pallas_call(
        functools.partial(_gemm_tile, n_inter=n_inter),
        grid_spec=grid_spec,
        out_shape=jax.ShapeDtypeStruct((p_pad, k_dim), jnp.float32),
    )(tile_expert, x_rows, w1, w2, row_weight)


def kernel(hidden_states, w1, w2, topk_weights, topk_ids):
    m, k_dim = hidden_states.shape
    e_dim = w1.shape[0]
    topk = topk_ids.shape[1]
    p = m * topk
    p_pad = p + e_dim * TM

    e_flat = topk_ids.reshape(p)
    onehot = (e_flat[None, :] == jnp.arange(e_dim, dtype=jnp.int32)[:, None])
    onehot = onehot.astype(jnp.int32)
    csum = jnp.cumsum(onehot, axis=1)
    counts = csum[:, -1]
    rank = jnp.sum(onehot * csum, axis=0) - 1
    pg = ((counts + TM - 1) // TM) * TM
    pg_cum = jnp.cumsum(pg)
    pstart = pg_cum - pg
    dest = (pstart[e_flat] + rank).astype(jnp.int32)
    row_token = (jnp.arange(p_pad, dtype=jnp.int32) % m).at[dest].set(
        (jnp.arange(p, dtype=jnp.int32) // topk))
    row_weight = jnp.zeros((p_pad, 1), jnp.float32).at[dest, 0].set(
        topk_weights.reshape(p))
    row_of_pair = dest
    tiles = p_pad // TM
    tile_expert = jnp.sum(
        (jnp.arange(tiles, dtype=jnp.int32)[:, None] * TM >= pg_cum[None, :])
        .astype(jnp.int32), axis=1)
    tile_expert = jnp.minimum(tile_expert, e_dim - 1).astype(jnp.int32)

    x_rows = _sc_dispatch(row_token, hidden_states)

    y_rows = _grouped_gemm(x_rows, w1, w2, row_weight, tile_expert)

    ridx = row_of_pair.reshape(m, topk).T
    out = _sc_finalize(ridx, y_rows, m)
    return out
